# baseline (speedup 1.0000x reference)
import functools

import jax
import jax.numpy as jnp
from jax import lax
from jax.experimental import pallas as pl
from jax.experimental.pallas import tpu as pltpu
from jax.experimental.pallas import tpu_sc as plsc

B = 16384
CTX = 6
D = 64
V = 1000000
L = 16
NC = 2
NS = 16
NW = NC * NS
RNG = 31232
GW = 384
NG = RNG // GW
NLOOK = B + B * CTX
DUMP = NLOOK
NROWS = NLOOK + 8
LCAP = 4608 + 16
SCAP = 384
XCHUNK = 24576

_mesh = plsc.VectorSubcoreMesh(
    core_axis_name="c", subcore_axis_name="s", num_cores=NC, num_subcores=NS
)


@functools.partial(
    pl.kernel,
    out_type=jax.ShapeDtypeStruct((NROWS, 128), jnp.float32),
    mesh=_mesh,
    scratch_types=[
        pltpu.VMEM((XCHUNK,), jnp.int32),
        pltpu.VMEM((LCAP,), jnp.int32),
        pltpu.VMEM((LCAP,), jnp.int32),
        pltpu.VMEM((512,), jnp.int32),
        pltpu.VMEM((512,), jnp.int32),
        pltpu.VMEM((D, GW), jnp.float32),
        pltpu.VMEM((D, 64), jnp.float32),
        pltpu.VMEM((SCAP, 128), jnp.float32),
        pltpu.VMEM((SCAP // 128, 128), jnp.int32),
        pltpu.SemaphoreType.DMA,
    ],
    compiler_params=pltpu.CompilerParams(needs_layout_passes=False),
)
def _extract(cidx_hbm, xidx_hbm, ctT_hbm, xtT_hbm, ctail_hbm, xtail_hbm,
             rows_hbm, idxb_v, lv_v, lg_v, gv_v, gg_v, slab_v, tail_v,
             stag_v, gid_v, sem):
    wid = lax.axis_index("s") * NC + lax.axis_index("c")
    vlo = wid * RNG
    vhi = jnp.where(wid == NW - 1, V, vlo + RNG)
    lanes = lax.iota(jnp.int32, L)
    big = jnp.full((L,), 0x7FFFFFFF, jnp.int32)
    dumpv = jnp.full((L,), DUMP, jnp.int32)

    def reset_gids():
        for r in range(SCAP // 128):
            for c in range(128 // L):
                gid_v[r, pl.ds(c * L, L)] = dumpv

    def scan_chunk(n_idx, gid_base, cnt0):
        def body(i, cnt):
            v = idxb_v[pl.ds(i * L, L)]
            m = (v >= vlo) & (v < vhi)
            pf = plsc.cumsum(m.astype(jnp.int32))
            pos = cnt + pf - 1
            plsc.store_scatter(lv_v, [pos], v, mask=m)
            plsc.store_scatter(lg_v, [pos], gid_base + i * L + lanes, mask=m)
            return cnt + jnp.sum(m.astype(jnp.int32))
        return lax.fori_loop(0, n_idx // L, body, cnt0)

    def flush(scnt):
        for r in range(SCAP // 128):
            @pl.when(scnt > r * 128)
            def _():
                pltpu.async_copy(
                    stag_v.at[pl.ds(r * 128, 128)],
                    rows_hbm.at[gid_v.at[r]],
                    sem,
                ).wait()
        reset_gids()

    def group_step(tab_hbm, glo, width, cnt, scnt, tail=False):
        @pl.when(scnt + 128 > SCAP)
        def _():
            flush(scnt)
        scnt = jnp.where(scnt + 128 > SCAP, 0, scnt)

        if tail:
            src_v = tail_v
            pltpu.sync_copy(tab_hbm, tail_v)
        else:
            src_v = slab_v
            pltpu.sync_copy(
                tab_hbm.at[:, pl.ds(glo, width)], slab_v.at[:, pl.ds(0, width)]
            )

        def rescan(i, gcnt):
            v = lv_v[pl.ds(i * L, L)]
            g = lg_v[pl.ds(i * L, L)]
            m = (v >= glo) & (v < glo + width)
            pf = plsc.cumsum(m.astype(jnp.int32))
            pos = gcnt + pf - 1
            plsc.store_scatter(gv_v, [pos], v - glo, mask=m)
            plsc.store_scatter(gg_v, [pos], g, mask=m)
            return gcnt + jnp.sum(m.astype(jnp.int32))

        gcnt = lax.fori_loop(0, (cnt + L - 1) // L, rescan, 0)

        def extract(j, _):
            valid = lanes < (gcnt - j * L)
            voff = jnp.where(valid, gv_v[pl.ds(j * L, L)], 0)
            gids = jnp.where(valid, gg_v[pl.ds(j * L, L)], dumpv)
            base = scnt + j * L
            for d in range(D):
                dsp = jnp.full((L,), d, jnp.int32)
                val = plsc.load_gather(src_v, [dsp, voff])
                plsc.store_scatter(stag_v, [base + lanes, dsp], val)
            gid_v[base // 128, pl.ds(base % 128, L)] = gids
            return 0

        nb = (gcnt + L - 1) // L
        lax.fori_loop(0, nb, extract, 0)
        return scnt + nb * L

    def run_pass(tab_hbm, tail_hbm, cnt):
        lv_v[pl.ds(cnt, L)] = big
        reset_gids()
        scnt = 0

        def grp(g, scnt):
            return group_step(tab_hbm, vlo + g * GW, GW, cnt, scnt)

        ngrp = jnp.where(wid == NW - 1, 82, NG)
        scnt = lax.fori_loop(0, ngrp, grp, jnp.int32(0))

        @pl.when(wid < NW - 1)
        def _():
            s2 = group_step(tab_hbm, vlo + NG * GW, 128, cnt, scnt)
            flush(s2)

        @pl.when(wid == NW - 1)
        def _():
            s2 = group_step(tab_hbm, vlo + 82 * GW, 256, cnt, scnt)
            s3 = group_step(tail_hbm, V - 64, 64, cnt, s2, tail=True)
            flush(s3)

    pltpu.sync_copy(cidx_hbm, idxb_v.at[pl.ds(0, B)])
    cnt = scan_chunk(B, 0, jnp.int32(0))
    run_pass(ctT_hbm, ctail_hbm, cnt)

    cnt = jnp.int32(0)
    for ch in range(B * CTX // XCHUNK):
        pltpu.sync_copy(xidx_hbm.at[pl.ds(ch * XCHUNK, XCHUNK)], idxb_v)
        cnt = scan_chunk(XCHUNK, B + ch * XCHUNK, cnt)
    run_pass(xtT_hbm, xtail_hbm, cnt)


RPW = B // NW
C2 = 64
NCH2 = RPW // C2
FPC = C2 * CTX
OPW = RPW * CTX


@functools.partial(
    pl.kernel,
    out_type=jax.ShapeDtypeStruct((B * CTX,), jnp.float32),
    mesh=_mesh,
    scratch_types=[
        pltpu.VMEM((C2, 128), jnp.float32),
        pltpu.VMEM((FPC, 128), jnp.float32),
        pltpu.VMEM((OPW,), jnp.int32),
        pltpu.VMEM((OPW,), jnp.float32),
        pltpu.SemaphoreType.DMA,
    ],
    compiler_params=pltpu.CompilerParams(needs_layout_passes=False),
)
def _dots(rows_hbm, mask_hbm, out_hbm, crows_v, xrows_v, mask_v, outb_v, sem):
    wid = lax.axis_index("s") * NC + lax.axis_index("c")
    obase = wid * OPW
    pltpu.sync_copy(mask_hbm.at[pl.ds(obase, OPW)], mask_v)
    lane = lax.iota(jnp.int32, L)
    GROUP = 8
    NVR = GROUP * CTX // L

    for chunk in range(NCH2):
        h1 = pltpu.async_copy(
            rows_hbm.at[pl.ds(wid * RPW + chunk * C2, C2)], crows_v, sem
        )
        h2 = pltpu.async_copy(
            rows_hbm.at[pl.ds(B + obase + chunk * FPC, FPC)], xrows_v, sem
        )
        h1.wait()
        h2.wait()
        dbase = chunk * FPC

        def dot_body(g, _):
            res = [jnp.zeros((L,), jnp.float32) for _ in range(NVR)]
            for r in range(GROUP):
                i = g * GROUP + r
                cvecs = [crows_v[i, pl.ds(k * L, L)] for k in range(D // L)]
                for j in range(CTX):
                    flat = i * CTX + j
                    pos = r * CTX + j
                    acc = cvecs[0] * xrows_v[flat, pl.ds(0, L)]
                    for k in range(1, D // L):
                        acc = acc + cvecs[k] * xrows_v[flat, pl.ds(k * L, L)]
                    s = jnp.sum(acc)
                    res[pos // L] = jnp.where(lane == (pos % L), s, res[pos // L])
            for t in range(NVR):
                m = mask_v[pl.ds(dbase + g * (GROUP * CTX) + t * L, L)]
                sig = 1.0 / (1.0 + jnp.exp(-res[t]))
                outb_v[pl.ds(dbase + g * (GROUP * CTX) + t * L, L)] = (
                    jnp.where(m == 0, 0.0, sig)
                )
            return 0

        lax.fori_loop(0, C2 // GROUP, dot_body, 0)

    pltpu.sync_copy(outb_v, out_hbm.at[pl.ds(obase, OPW)])


def kernel(center, context, mask, center_table, context_table):
    ctT = center_table.T
    xtT = context_table.T
    rows = _extract(
        center.reshape(-1),
        context.reshape(-1),
        ctT,
        xtT,
        ctT[:, V - 64:],
        xtT[:, V - 64:],
    )
    out = _dots(rows, mask.reshape(-1))
    return out.reshape(B, CTX)


# speedup vs baseline: 1.0013x; 1.0013x over previous
import functools

import jax
import jax.numpy as jnp
from jax import lax
from jax.experimental import pallas as pl
from jax.experimental.pallas import tpu as pltpu
from jax.experimental.pallas import tpu_sc as plsc

B = 16384
CTX = 6
D = 64
V = 1000000
L = 16
NC = 2
NS = 16
NW = NC * NS
RNG = 31232
GW = 384
NG = RNG // GW
NLOOK = B + B * CTX
DUMP = NLOOK
NROWS = NLOOK + 8
LCAP = 4608 + 80
SCAP = 384
XCHUNK = 24576

_mesh = plsc.VectorSubcoreMesh(
    core_axis_name="c", subcore_axis_name="s", num_cores=NC, num_subcores=NS
)


@functools.partial(
    pl.kernel,
    out_type=jax.ShapeDtypeStruct((NROWS, 128), jnp.float32),
    mesh=_mesh,
    scratch_types=[
        pltpu.VMEM((XCHUNK,), jnp.int32),
        pltpu.VMEM((LCAP,), jnp.int32),
        pltpu.VMEM((LCAP,), jnp.int32),
        pltpu.VMEM((512,), jnp.int32),
        pltpu.VMEM((512,), jnp.int32),
        pltpu.VMEM((D, GW), jnp.float32),
        pltpu.VMEM((D, 64), jnp.float32),
        pltpu.VMEM((SCAP, 128), jnp.float32),
        pltpu.VMEM((SCAP // 128, 128), jnp.int32),
        pltpu.SemaphoreType.DMA,
    ],
    compiler_params=pltpu.CompilerParams(needs_layout_passes=False),
)
def _extract(cidx_hbm, xidx_hbm, ctT_hbm, xtT_hbm, ctail_hbm, xtail_hbm,
             rows_hbm, idxb_v, lv_v, lg_v, gv_v, gg_v, slab_v, tail_v,
             stag_v, gid_v, sem):
    wid = lax.axis_index("s") * NC + lax.axis_index("c")
    vlo = wid * RNG
    vhi = jnp.where(wid == NW - 1, V, vlo + RNG)
    lanes = lax.iota(jnp.int32, L)
    big = jnp.full((L,), 0x7FFFFFFF, jnp.int32)
    dumpv = jnp.full((L,), DUMP, jnp.int32)

    def reset_gids():
        for r in range(SCAP // 128):
            for c in range(128 // L):
                gid_v[r, pl.ds(c * L, L)] = dumpv

    def scan_chunk(n_idx, gid_base, cnt0):
        def body(i, cnt):
            v = idxb_v[pl.ds(i * L, L)]
            m = (v >= vlo) & (v < vhi)
            pf = plsc.cumsum(m.astype(jnp.int32))
            pos = cnt + pf - 1
            plsc.store_scatter(lv_v, [pos], v, mask=m)
            plsc.store_scatter(lg_v, [pos], gid_base + i * L + lanes, mask=m)
            return cnt + pf[L - 1]
        return lax.fori_loop(0, n_idx // L, body, cnt0, unroll=4)

    def flush(scnt):
        for r in range(SCAP // 128):
            @pl.when(scnt > r * 128)
            def _():
                pltpu.async_copy(
                    stag_v.at[pl.ds(r * 128, 128)],
                    rows_hbm.at[gid_v.at[r]],
                    sem,
                ).wait()
        reset_gids()

    def group_step(tab_hbm, glo, width, cnt, scnt, tail=False):
        @pl.when(scnt + 128 > SCAP)
        def _():
            flush(scnt)
        scnt = jnp.where(scnt + 128 > SCAP, 0, scnt)

        if tail:
            src_v = tail_v
            pltpu.sync_copy(tab_hbm, tail_v)
        else:
            src_v = slab_v
            pltpu.sync_copy(
                tab_hbm.at[:, pl.ds(glo, width)], slab_v.at[:, pl.ds(0, width)]
            )

        def rescan(i, gcnt):
            for u in range(4):
                v = lv_v[pl.ds((i * 4 + u) * L, L)]
                g = lg_v[pl.ds((i * 4 + u) * L, L)]
                m = (v >= glo) & (v < glo + width)
                pf = plsc.cumsum(m.astype(jnp.int32))
                pos = gcnt + pf - 1
                plsc.store_scatter(gv_v, [pos], v - glo, mask=m)
                plsc.store_scatter(gg_v, [pos], g, mask=m)
                gcnt = gcnt + pf[L - 1]
            return gcnt

        gcnt = lax.fori_loop(0, (cnt + 4 * L - 1) // (4 * L), rescan, 0)

        def extract(j, _):
            valid = lanes < (gcnt - j * L)
            voff = jnp.where(valid, gv_v[pl.ds(j * L, L)], 0)
            gids = jnp.where(valid, gg_v[pl.ds(j * L, L)], dumpv)
            base = scnt + j * L
            rowix = base + lanes
            for d in range(D):
                dsp = jnp.full((L,), d, jnp.int32)
                val = plsc.load_gather(src_v, [dsp, voff])
                plsc.store_scatter(stag_v, [rowix, dsp], val)
            gid_v[base // 128, pl.ds(base % 128, L)] = gids
            return 0

        nb = (gcnt + L - 1) // L
        lax.fori_loop(0, nb, extract, 0)
        return scnt + nb * L

    def run_pass(tab_hbm, tail_hbm, cnt):
        for u in range(4):
            lv_v[pl.ds(cnt + u * L, L)] = big
        reset_gids()
        scnt = 0

        def grp(g, scnt):
            return group_step(tab_hbm, vlo + g * GW, GW, cnt, scnt)

        ngrp = jnp.where(wid == NW - 1, 82, NG)
        scnt = lax.fori_loop(0, ngrp, grp, jnp.int32(0))

        @pl.when(wid < NW - 1)
        def _():
            s2 = group_step(tab_hbm, vlo + NG * GW, 128, cnt, scnt)
            flush(s2)

        @pl.when(wid == NW - 1)
        def _():
            s2 = group_step(tab_hbm, vlo + 82 * GW, 256, cnt, scnt)
            s3 = group_step(tail_hbm, V - 64, 64, cnt, s2, tail=True)
            flush(s3)

    pltpu.sync_copy(cidx_hbm, idxb_v.at[pl.ds(0, B)])
    cnt = scan_chunk(B, 0, jnp.int32(0))
    run_pass(ctT_hbm, ctail_hbm, cnt)

    cnt = jnp.int32(0)
    for ch in range(B * CTX // XCHUNK):
        pltpu.sync_copy(xidx_hbm.at[pl.ds(ch * XCHUNK, XCHUNK)], idxb_v)
        cnt = scan_chunk(XCHUNK, B + ch * XCHUNK, cnt)
    run_pass(xtT_hbm, xtail_hbm, cnt)


RPW = B // NW
C2 = 64
NCH2 = RPW // C2
FPC = C2 * CTX
OPW = RPW * CTX


@functools.partial(
    pl.kernel,
    out_type=jax.ShapeDtypeStruct((B * CTX,), jnp.float32),
    mesh=_mesh,
    scratch_types=[
        pltpu.VMEM((C2, 128), jnp.float32),
        pltpu.VMEM((FPC, 128), jnp.float32),
        pltpu.VMEM((OPW,), jnp.int32),
        pltpu.VMEM((OPW,), jnp.float32),
        pltpu.SemaphoreType.DMA,
    ],
    compiler_params=pltpu.CompilerParams(needs_layout_passes=False),
)
def _dots(rows_hbm, mask_hbm, out_hbm, crows_v, xrows_v, mask_v, outb_v, sem):
    wid = lax.axis_index("s") * NC + lax.axis_index("c")
    obase = wid * OPW
    pltpu.sync_copy(mask_hbm.at[pl.ds(obase, OPW)], mask_v)
    lane = lax.iota(jnp.int32, L)
    GROUP = 8
    NVR = GROUP * CTX // L

    for chunk in range(NCH2):
        h1 = pltpu.async_copy(
            rows_hbm.at[pl.ds(wid * RPW + chunk * C2, C2)], crows_v, sem
        )
        h2 = pltpu.async_copy(
            rows_hbm.at[pl.ds(B + obase + chunk * FPC, FPC)], xrows_v, sem
        )
        h1.wait()
        h2.wait()
        dbase = chunk * FPC

        def dot_body(g, _):
            res = [jnp.zeros((L,), jnp.float32) for _ in range(NVR)]
            for r in range(GROUP):
                i = g * GROUP + r
                cvecs = [crows_v[i, pl.ds(k * L, L)] for k in range(D // L)]
                for j in range(CTX):
                    flat = i * CTX + j
                    pos = r * CTX + j
                    acc = cvecs[0] * xrows_v[flat, pl.ds(0, L)]
                    for k in range(1, D // L):
                        acc = acc + cvecs[k] * xrows_v[flat, pl.ds(k * L, L)]
                    s = jnp.sum(acc)
                    res[pos // L] = jnp.where(lane == (pos % L), s, res[pos // L])
            for t in range(NVR):
                m = mask_v[pl.ds(dbase + g * (GROUP * CTX) + t * L, L)]
                sig = 1.0 / (1.0 + jnp.exp(-res[t]))
                outb_v[pl.ds(dbase + g * (GROUP * CTX) + t * L, L)] = (
                    jnp.where(m == 0, 0.0, sig)
                )
            return 0

        lax.fori_loop(0, C2 // GROUP, dot_body, 0)

    pltpu.sync_copy(outb_v, out_hbm.at[pl.ds(obase, OPW)])


def kernel(center, context, mask, center_table, context_table):
    ctT = center_table.T
    xtT = context_table.T
    rows = _extract(
        center.reshape(-1),
        context.reshape(-1),
        ctT,
        xtT,
        ctT[:, V - 64:],
        xtT[:, V - 64:],
    )
    out = _dots(rows, mask.reshape(-1))
    return out.reshape(B, CTX)


# speedup vs baseline: 1.0021x; 1.0008x over previous
import functools

import jax
import jax.numpy as jnp
from jax import lax
from jax.experimental import pallas as pl
from jax.experimental.pallas import tpu as pltpu
from jax.experimental.pallas import tpu_sc as plsc

B = 16384
CTX = 6
D = 64
V = 1000000
L = 16
NC = 2
NS = 16
NW = NC * NS
RNG = 31232
GW = 384
NG = RNG // GW
NLOOK = B + B * CTX
DUMP = NLOOK
NROWS = NLOOK + 8
LCAP = 4608 + 80
SCAP = 384
XCHUNK = 24576

_mesh = plsc.VectorSubcoreMesh(
    core_axis_name="c", subcore_axis_name="s", num_cores=NC, num_subcores=NS
)


@functools.partial(
    pl.kernel,
    out_type=jax.ShapeDtypeStruct((NROWS, 128), jnp.float32),
    mesh=_mesh,
    scratch_types=[
        pltpu.VMEM((XCHUNK,), jnp.int32),
        pltpu.VMEM((LCAP,), jnp.int32),
        pltpu.VMEM((LCAP,), jnp.int32),
        pltpu.VMEM((512,), jnp.int32),
        pltpu.VMEM((512,), jnp.int32),
        pltpu.VMEM((D, GW), jnp.float32),
        pltpu.VMEM((D, 64), jnp.float32),
        pltpu.VMEM((SCAP, 128), jnp.float32),
        pltpu.VMEM((SCAP // 128, 128), jnp.int32),
        pltpu.SemaphoreType.DMA,
    ],
    compiler_params=pltpu.CompilerParams(needs_layout_passes=False),
)
def _extract(cidx_hbm, xidx_hbm, ctT_hbm, xtT_hbm, ctail_hbm, xtail_hbm,
             rows_hbm, idxb_v, lv_v, lg_v, gv_v, gg_v, slab_v, tail_v,
             stag_v, gid_v, sem):
    wid = lax.axis_index("s") * NC + lax.axis_index("c")
    vlo = wid * RNG
    vhi = jnp.where(wid == NW - 1, V, vlo + RNG)
    lanes = lax.iota(jnp.int32, L)
    big = jnp.full((L,), 0x7FFFFFFF, jnp.int32)
    dumpv = jnp.full((L,), DUMP, jnp.int32)

    def reset_gids():
        for r in range(SCAP // 128):
            for c in range(128 // L):
                gid_v[r, pl.ds(c * L, L)] = dumpv

    def scan_chunk(n_idx, gid_base, cnt0):
        def body(i, cnt):
            v = idxb_v[pl.ds(i * L, L)]
            m = (v >= vlo) & (v < vhi)
            pf = plsc.cumsum(m.astype(jnp.int32))
            pos = cnt + pf - 1
            plsc.store_scatter(lv_v, [pos], v, mask=m)
            plsc.store_scatter(lg_v, [pos], gid_base + i * L + lanes, mask=m)
            return cnt + pf[L - 1]
        return lax.fori_loop(0, n_idx // L, body, cnt0, unroll=4)

    def flush(scnt):
        for r in range(SCAP // 128):
            @pl.when(scnt > r * 128)
            def _():
                pltpu.async_copy(
                    stag_v.at[pl.ds(r * 128, 128)],
                    rows_hbm.at[gid_v.at[r]],
                    sem,
                ).wait()
        reset_gids()

    def group_step(tab_hbm, glo, width, cnt, scnt, tail=False):
        @pl.when(scnt + 128 > SCAP)
        def _():
            flush(scnt)
        scnt = jnp.where(scnt + 128 > SCAP, 0, scnt)

        if tail:
            src_v = tail_v
            pltpu.sync_copy(tab_hbm, tail_v)
        else:
            src_v = slab_v
            hs = [
                pltpu.async_copy(
                    tab_hbm.at[pl.ds(8 * tr, 8), pl.ds(glo, width)],
                    slab_v.at[pl.ds(8 * tr, 8), pl.ds(0, width)],
                    sem,
                )
                for tr in range(D // 8)
            ]
            for h in hs:
                h.wait()

        def rescan(i, gcnt):
            for u in range(4):
                v = lv_v[pl.ds((i * 4 + u) * L, L)]
                g = lg_v[pl.ds((i * 4 + u) * L, L)]
                m = (v >= glo) & (v < glo + width)
                pf = plsc.cumsum(m.astype(jnp.int32))
                pos = gcnt + pf - 1
                plsc.store_scatter(gv_v, [pos], v - glo, mask=m)
                plsc.store_scatter(gg_v, [pos], g, mask=m)
                gcnt = gcnt + pf[L - 1]
            return gcnt

        gcnt = lax.fori_loop(0, (cnt + 4 * L - 1) // (4 * L), rescan, 0)

        def extract(j, _):
            valid = lanes < (gcnt - j * L)
            voff = jnp.where(valid, gv_v[pl.ds(j * L, L)], 0)
            gids = jnp.where(valid, gg_v[pl.ds(j * L, L)], dumpv)
            base = scnt + j * L
            rowix = base + lanes
            for d in range(D):
                dsp = jnp.full((L,), d, jnp.int32)
                val = plsc.load_gather(src_v, [dsp, voff])
                plsc.store_scatter(stag_v, [rowix, dsp], val)
            gid_v[base // 128, pl.ds(base % 128, L)] = gids
            return 0

        nb = (gcnt + L - 1) // L
        lax.fori_loop(0, nb, extract, 0)
        return scnt + nb * L

    def run_pass(tab_hbm, tail_hbm, cnt):
        for u in range(4):
            lv_v[pl.ds(cnt + u * L, L)] = big
        reset_gids()
        scnt = 0

        def grp(g, scnt):
            return group_step(tab_hbm, vlo + g * GW, GW, cnt, scnt)

        ngrp = jnp.where(wid == NW - 1, 82, NG)
        scnt = lax.fori_loop(0, ngrp, grp, jnp.int32(0))

        @pl.when(wid < NW - 1)
        def _():
            s2 = group_step(tab_hbm, vlo + NG * GW, 128, cnt, scnt)
            flush(s2)

        @pl.when(wid == NW - 1)
        def _():
            s2 = group_step(tab_hbm, vlo + 82 * GW, 256, cnt, scnt)
            s3 = group_step(tail_hbm, V - 64, 64, cnt, s2, tail=True)
            flush(s3)

    pltpu.sync_copy(cidx_hbm, idxb_v.at[pl.ds(0, B)])
    cnt = scan_chunk(B, 0, jnp.int32(0))
    run_pass(ctT_hbm, ctail_hbm, cnt)

    cnt = jnp.int32(0)
    for ch in range(B * CTX // XCHUNK):
        pltpu.sync_copy(xidx_hbm.at[pl.ds(ch * XCHUNK, XCHUNK)], idxb_v)
        cnt = scan_chunk(XCHUNK, B + ch * XCHUNK, cnt)
    run_pass(xtT_hbm, xtail_hbm, cnt)


RPW = B // NW
C2 = 64
NCH2 = RPW // C2
FPC = C2 * CTX
OPW = RPW * CTX


@functools.partial(
    pl.kernel,
    out_type=jax.ShapeDtypeStruct((B * CTX,), jnp.float32),
    mesh=_mesh,
    scratch_types=[
        pltpu.VMEM((C2, 128), jnp.float32),
        pltpu.VMEM((FPC, 128), jnp.float32),
        pltpu.VMEM((OPW,), jnp.int32),
        pltpu.VMEM((OPW,), jnp.float32),
        pltpu.SemaphoreType.DMA,
    ],
    compiler_params=pltpu.CompilerParams(needs_layout_passes=False),
)
def _dots(rows_hbm, mask_hbm, out_hbm, crows_v, xrows_v, mask_v, outb_v, sem):
    wid = lax.axis_index("s") * NC + lax.axis_index("c")
    obase = wid * OPW
    pltpu.sync_copy(mask_hbm.at[pl.ds(obase, OPW)], mask_v)
    lane = lax.iota(jnp.int32, L)
    GROUP = 8
    NVR = GROUP * CTX // L

    for chunk in range(NCH2):
        h1 = pltpu.async_copy(
            rows_hbm.at[pl.ds(wid * RPW + chunk * C2, C2)], crows_v, sem
        )
        h2 = pltpu.async_copy(
            rows_hbm.at[pl.ds(B + obase + chunk * FPC, FPC)], xrows_v, sem
        )
        h1.wait()
        h2.wait()
        dbase = chunk * FPC

        def dot_body(g, _):
            res = [jnp.zeros((L,), jnp.float32) for _ in range(NVR)]
            for r in range(GROUP):
                i = g * GROUP + r
                cvecs = [crows_v[i, pl.ds(k * L, L)] for k in range(D // L)]
                for j in range(CTX):
                    flat = i * CTX + j
                    pos = r * CTX + j
                    acc = cvecs[0] * xrows_v[flat, pl.ds(0, L)]
                    for k in range(1, D // L):
                        acc = acc + cvecs[k] * xrows_v[flat, pl.ds(k * L, L)]
                    s = jnp.sum(acc)
                    res[pos // L] = jnp.where(lane == (pos % L), s, res[pos // L])
            for t in range(NVR):
                m = mask_v[pl.ds(dbase + g * (GROUP * CTX) + t * L, L)]
                sig = 1.0 / (1.0 + jnp.exp(-res[t]))
                outb_v[pl.ds(dbase + g * (GROUP * CTX) + t * L, L)] = (
                    jnp.where(m == 0, 0.0, sig)
                )
            return 0

        lax.fori_loop(0, C2 // GROUP, dot_body, 0)

    pltpu.sync_copy(outb_v, out_hbm.at[pl.ds(obase, OPW)])


def kernel(center, context, mask, center_table, context_table):
    ctT = center_table.T
    xtT = context_table.T
    rows = _extract(
        center.reshape(-1),
        context.reshape(-1),
        ctT,
        xtT,
        ctT[:, V - 64:],
        xtT[:, V - 64:],
    )
    out = _dots(rows, mask.reshape(-1))
    return out.reshape(B, CTX)


# speedup vs baseline: 4.8703x; 4.8601x over previous
import functools

import jax
import jax.numpy as jnp
from jax import lax
from jax.experimental import pallas as pl
from jax.experimental.pallas import tpu as pltpu
from jax.experimental.pallas import tpu_sc as plsc

B = 16384
CTX = 6
D = 64
V = 1000000
L = 16
NC = 2
NS = 16
NW = NC * NS
RNG = 31232
GW = 384
NG = RNG // GW
NLOOK = B + B * CTX
NROWS = NLOOK + NW * 384 + 8
LCAP = 4608 + 80
SCAP = 384
XCHUNK = 24576

_mesh = plsc.VectorSubcoreMesh(
    core_axis_name="c", subcore_axis_name="s", num_cores=NC, num_subcores=NS
)


@functools.partial(
    pl.kernel,
    out_type=jax.ShapeDtypeStruct((NROWS, 128), jnp.float32),
    mesh=_mesh,
    scratch_types=[
        pltpu.VMEM((XCHUNK,), jnp.int32),
        pltpu.VMEM((LCAP,), jnp.int32),
        pltpu.VMEM((LCAP,), jnp.int32),
        pltpu.VMEM((512,), jnp.int32),
        pltpu.VMEM((512,), jnp.int32),
        pltpu.VMEM((D, GW), jnp.float32),
        pltpu.VMEM((D, 64), jnp.float32),
        pltpu.VMEM((SCAP, 128), jnp.float32),
        pltpu.VMEM((SCAP // 128, 128), jnp.int32),
        pltpu.SemaphoreType.DMA,
    ],
    compiler_params=pltpu.CompilerParams(needs_layout_passes=False),
)
def _extract(cidx_hbm, xidx_hbm, ctT_hbm, xtT_hbm, ctail_hbm, xtail_hbm,
             rows_hbm, idxb_v, lv_v, lg_v, gv_v, gg_v, slab_v, tail_v,
             stag_v, gid_v, sem):
    wid = lax.axis_index("s") * NC + lax.axis_index("c")
    vlo = wid * RNG
    vhi = jnp.where(wid == NW - 1, V, vlo + RNG)
    lanes = lax.iota(jnp.int32, L)
    big = jnp.full((L,), 0x7FFFFFFF, jnp.int32)
    dumpbase = NLOOK + wid * SCAP

    def reset_gids():
        for r in range(SCAP // 128):
            for c in range(128 // L):
                gid_v[r, pl.ds(c * L, L)] = dumpbase + r * 128 + c * L + lanes

    def scan_chunk(n_idx, gid_base, cnt0):
        def body(i, cnt):
            v = idxb_v[pl.ds(i * L, L)]
            m = (v >= vlo) & (v < vhi)
            pf = plsc.cumsum(m.astype(jnp.int32))
            pos = cnt + pf - 1
            plsc.store_scatter(lv_v, [pos], v, mask=m)
            plsc.store_scatter(lg_v, [pos], gid_base + i * L + lanes, mask=m)
            return cnt + pf[L - 1]
        return lax.fori_loop(0, n_idx // L, body, cnt0, unroll=4)

    def flush(scnt):
        for r in range(SCAP // 128):
            @pl.when(scnt > r * 128)
            def _():
                pltpu.async_copy(
                    stag_v.at[pl.ds(r * 128, 128)],
                    rows_hbm.at[gid_v.at[r]],
                    sem,
                ).wait()
        reset_gids()

    def group_step(tab_hbm, glo, width, cnt, scnt, tail=False):
        @pl.when(scnt + 128 > SCAP)
        def _():
            flush(scnt)
        scnt = jnp.where(scnt + 128 > SCAP, 0, scnt)

        if tail:
            src_v = tail_v
            pltpu.sync_copy(tab_hbm, tail_v)
        else:
            src_v = slab_v
            hs = [
                pltpu.async_copy(
                    tab_hbm.at[pl.ds(8 * tr, 8), pl.ds(glo, width)],
                    slab_v.at[pl.ds(8 * tr, 8), pl.ds(0, width)],
                    sem,
                )
                for tr in range(D // 8)
            ]
            for h in hs:
                h.wait()

        def rescan(i, gcnt):
            for u in range(4):
                v = lv_v[pl.ds((i * 4 + u) * L, L)]
                g = lg_v[pl.ds((i * 4 + u) * L, L)]
                m = (v >= glo) & (v < glo + width)
                pf = plsc.cumsum(m.astype(jnp.int32))
                pos = gcnt + pf - 1
                plsc.store_scatter(gv_v, [pos], v - glo, mask=m)
                plsc.store_scatter(gg_v, [pos], g, mask=m)
                gcnt = gcnt + pf[L - 1]
            return gcnt

        gcnt = lax.fori_loop(0, (cnt + 4 * L - 1) // (4 * L), rescan, 0)

        def extract(j, _):
            valid = lanes < (gcnt - j * L)
            voff = jnp.where(valid, gv_v[pl.ds(j * L, L)], 0)
            base = scnt + j * L
            rowix = base + lanes
            gids = jnp.where(valid, gg_v[pl.ds(j * L, L)], dumpbase + rowix)
            for d in range(D):
                dsp = jnp.full((L,), d, jnp.int32)
                val = plsc.load_gather(src_v, [dsp, voff])
                plsc.store_scatter(stag_v, [rowix, dsp], val)
            gid_v[base // 128, pl.ds(base % 128, L)] = gids
            return 0

        nb = (gcnt + L - 1) // L
        lax.fori_loop(0, nb, extract, 0)
        return scnt + nb * L

    def run_pass(tab_hbm, tail_hbm, cnt):
        for u in range(4):
            lv_v[pl.ds(cnt + u * L, L)] = big
        reset_gids()
        scnt = 0

        def grp(g, scnt):
            return group_step(tab_hbm, vlo + g * GW, GW, cnt, scnt)

        ngrp = jnp.where(wid == NW - 1, 82, NG)
        scnt = lax.fori_loop(0, ngrp, grp, jnp.int32(0))

        @pl.when(wid < NW - 1)
        def _():
            s2 = group_step(tab_hbm, vlo + NG * GW, 128, cnt, scnt)
            flush(s2)

        @pl.when(wid == NW - 1)
        def _():
            s2 = group_step(tab_hbm, vlo + 82 * GW, 256, cnt, scnt)
            s3 = group_step(tail_hbm, V - 64, 64, cnt, s2, tail=True)
            flush(s3)

    pltpu.sync_copy(cidx_hbm, idxb_v.at[pl.ds(0, B)])
    cnt = scan_chunk(B, 0, jnp.int32(0))
    run_pass(ctT_hbm, ctail_hbm, cnt)

    cnt = jnp.int32(0)
    for ch in range(B * CTX // XCHUNK):
        pltpu.sync_copy(xidx_hbm.at[pl.ds(ch * XCHUNK, XCHUNK)], idxb_v)
        cnt = scan_chunk(XCHUNK, B + ch * XCHUNK, cnt)
    run_pass(xtT_hbm, xtail_hbm, cnt)


RPW = B // NW
C2 = 64
NCH2 = RPW // C2
FPC = C2 * CTX
OPW = RPW * CTX


@functools.partial(
    pl.kernel,
    out_type=jax.ShapeDtypeStruct((B * CTX,), jnp.float32),
    mesh=_mesh,
    scratch_types=[
        pltpu.VMEM((C2, 128), jnp.float32),
        pltpu.VMEM((FPC, 128), jnp.float32),
        pltpu.VMEM((OPW,), jnp.int32),
        pltpu.VMEM((OPW,), jnp.float32),
        pltpu.SemaphoreType.DMA,
    ],
    compiler_params=pltpu.CompilerParams(needs_layout_passes=False),
)
def _dots(rows_hbm, mask_hbm, out_hbm, crows_v, xrows_v, mask_v, outb_v, sem):
    wid = lax.axis_index("s") * NC + lax.axis_index("c")
    obase = wid * OPW
    pltpu.sync_copy(mask_hbm.at[pl.ds(obase, OPW)], mask_v)
    lane = lax.iota(jnp.int32, L)
    GROUP = 8
    NVR = GROUP * CTX // L

    for chunk in range(NCH2):
        h1 = pltpu.async_copy(
            rows_hbm.at[pl.ds(wid * RPW + chunk * C2, C2)], crows_v, sem
        )
        h2 = pltpu.async_copy(
            rows_hbm.at[pl.ds(B + obase + chunk * FPC, FPC)], xrows_v, sem
        )
        h1.wait()
        h2.wait()
        dbase = chunk * FPC

        def dot_body(g, _):
            res = [jnp.zeros((L,), jnp.float32) for _ in range(NVR)]
            for r in range(GROUP):
                i = g * GROUP + r
                cvecs = [crows_v[i, pl.ds(k * L, L)] for k in range(D // L)]
                for j in range(CTX):
                    flat = i * CTX + j
                    pos = r * CTX + j
                    acc = cvecs[0] * xrows_v[flat, pl.ds(0, L)]
                    for k in range(1, D // L):
                        acc = acc + cvecs[k] * xrows_v[flat, pl.ds(k * L, L)]
                    s = jnp.sum(acc)
                    res[pos // L] = jnp.where(lane == (pos % L), s, res[pos // L])
            for t in range(NVR):
                m = mask_v[pl.ds(dbase + g * (GROUP * CTX) + t * L, L)]
                sig = 1.0 / (1.0 + jnp.exp(-res[t]))
                outb_v[pl.ds(dbase + g * (GROUP * CTX) + t * L, L)] = (
                    jnp.where(m == 0, 0.0, sig)
                )
            return 0

        lax.fori_loop(0, C2 // GROUP, dot_body, 0)

    pltpu.sync_copy(outb_v, out_hbm.at[pl.ds(obase, OPW)])


def kernel(center, context, mask, center_table, context_table):
    ctT = center_table.T
    xtT = context_table.T
    rows = _extract(
        center.reshape(-1),
        context.reshape(-1),
        ctT,
        xtT,
        ctT[:, V - 64:],
        xtT[:, V - 64:],
    )
    out = _dots(rows, mask.reshape(-1))
    return out.reshape(B, CTX)


# speedup vs baseline: 6.5278x; 1.3403x over previous
import functools

import jax
import jax.numpy as jnp
from jax import lax
from jax.experimental import pallas as pl
from jax.experimental.pallas import tpu as pltpu
from jax.experimental.pallas import tpu_sc as plsc

B = 16384
CTX = 6
D = 64
V = 1000000
L = 16
NC = 2
NS = 16
NW = NC * NS
RNG = 31232
GW = 512
NG = RNG // GW
NBKT = 8
BCAP = 704
NLOOK = B + B * CTX
NROWS = NLOOK + NW * 384 + 8
LCAP = 4608 + 80
SCAP = 384
XCHUNK = 16384

_mesh = plsc.VectorSubcoreMesh(
    core_axis_name="c", subcore_axis_name="s", num_cores=NC, num_subcores=NS
)


@functools.partial(
    pl.kernel,
    out_type=jax.ShapeDtypeStruct((NROWS, 128), jnp.float32),
    mesh=_mesh,
    scratch_types=[
        pltpu.VMEM((XCHUNK,), jnp.int32),
        pltpu.VMEM((LCAP,), jnp.int32),
        pltpu.VMEM((LCAP,), jnp.int32),
        pltpu.VMEM((NBKT * BCAP,), jnp.int32),
        pltpu.VMEM((NBKT * BCAP,), jnp.int32),
        pltpu.VMEM((512,), jnp.int32),
        pltpu.VMEM((512,), jnp.int32),
        pltpu.VMEM((D, GW), jnp.float32),
        pltpu.VMEM((D, 64), jnp.float32),
        pltpu.VMEM((SCAP, 128), jnp.float32),
        pltpu.VMEM((SCAP // 128, 128), jnp.int32),
        pltpu.SemaphoreType.DMA,
    ],
    compiler_params=pltpu.CompilerParams(needs_layout_passes=False),
)
def _extract(cidx_hbm, xidx_hbm, ctT_hbm, xtT_hbm, ctail_hbm, xtail_hbm,
             rows_hbm, idxb_v, lv_v, lg_v, bv_v, bg_v, gv_v, gg_v, slab_v,
             tail_v, stag_v, gid_v, sem):
    wid = lax.axis_index("s") * NC + lax.axis_index("c")
    vlo = wid * RNG
    vhi = jnp.where(wid == NW - 1, V, vlo + RNG)
    lanes = lax.iota(jnp.int32, L)
    big = jnp.full((L,), 0x7FFFFFFF, jnp.int32)
    dumpbase = NLOOK + wid * SCAP

    def reset_gids():
        for r in range(SCAP // 128):
            for c in range(128 // L):
                gid_v[r, pl.ds(c * L, L)] = dumpbase + r * 128 + c * L + lanes

    def scan_chunk(n_idx, gid_base, cnt0):
        def body(i, cnt):
            v = idxb_v[pl.ds(i * L, L)]
            m = (v >= vlo) & (v < vhi)
            pf = plsc.cumsum(m.astype(jnp.int32))
            pos = cnt + pf - 1
            plsc.store_scatter(lv_v, [pos], v, mask=m)
            plsc.store_scatter(lg_v, [pos], gid_base + i * L + lanes, mask=m)
            return cnt + pf[L - 1]
        return lax.fori_loop(0, n_idx // L, body, cnt0, unroll=4)

    def flush(scnt):
        for r in range(SCAP // 128):
            @pl.when(scnt > r * 128)
            def _():
                pltpu.async_copy(
                    stag_v.at[pl.ds(r * 128, 128)],
                    rows_hbm.at[gid_v.at[r]],
                    sem,
                ).wait()
        reset_gids()

    def group_step(tab_hbm, glo, width, bkoff, cntg, scnt, tail=False):
        @pl.when(scnt + 128 > SCAP)
        def _():
            flush(scnt)
        scnt = jnp.where(scnt + 128 > SCAP, 0, scnt)

        if tail:
            src_v = tail_v
            pltpu.sync_copy(tab_hbm, tail_v)
        else:
            src_v = slab_v
            hs = [
                pltpu.async_copy(
                    tab_hbm.at[pl.ds(8 * tr, 8), pl.ds(glo, width)],
                    slab_v.at[pl.ds(8 * tr, 8), pl.ds(0, width)],
                    sem,
                )
                for tr in range(D // 8)
            ]
            for h in hs:
                h.wait()

        def rescan(i, gcnt):
            for u in range(4):
                off = bkoff + (i * 4 + u) * L
                v = bv_v[pl.ds(off, L)]
                g = bg_v[pl.ds(off, L)]
                m = (v >= glo) & (v < glo + width)
                pf = plsc.cumsum(m.astype(jnp.int32))
                pos = gcnt + pf - 1
                plsc.store_scatter(gv_v, [pos], v - glo, mask=m)
                plsc.store_scatter(gg_v, [pos], g, mask=m)
                gcnt = gcnt + pf[L - 1]
            return gcnt

        gcnt = lax.fori_loop(0, (cntg + 4 * L - 1) // (4 * L), rescan, 0)

        def extract(j, _):
            valid = lanes < (gcnt - j * L)
            voff = jnp.where(valid, gv_v[pl.ds(j * L, L)], 0)
            base = scnt + j * L
            rowix = base + lanes
            gids = jnp.where(valid, gg_v[pl.ds(j * L, L)], dumpbase + rowix)
            for d in range(D):
                dsp = jnp.full((L,), d, jnp.int32)
                val = plsc.load_gather(src_v, [dsp, voff])
                plsc.store_scatter(stag_v, [rowix, dsp], val)
            gid_v[base // 128, pl.ds(base % 128, L)] = gids
            return 0

        nb = (gcnt + L - 1) // L
        lax.fori_loop(0, nb, extract, 0)
        return scnt + nb * L

    def run_pass(tab_hbm, tail_hbm, cnt):
        for u in range(4):
            lv_v[pl.ds(cnt + u * L, L)] = big
        reset_gids()

        bkcnt_vec = jnp.zeros((L,), jnp.int32)
        nvr4 = (cnt + 4 * L - 1) // (4 * L)
        for bk in range(NBKT):
            blo = vlo + bk * 4096

            def bf(i, c, blo=blo, bk=bk):
                for u in range(4):
                    v = lv_v[pl.ds((i * 4 + u) * L, L)]
                    g = lg_v[pl.ds((i * 4 + u) * L, L)]
                    m = (v >= blo) & (v < blo + 4096)
                    pf = plsc.cumsum(m.astype(jnp.int32))
                    pos = bk * BCAP + c + pf - 1
                    plsc.store_scatter(bv_v, [pos], v, mask=m)
                    plsc.store_scatter(bg_v, [pos], g, mask=m)
                    c = c + pf[L - 1]
                return c

            c = lax.fori_loop(0, nvr4, bf, jnp.int32(0))
            for u in range(4):
                bv_v[pl.ds(bk * BCAP + c + u * L, L)] = big
            bkcnt_vec = jnp.where(lanes == bk, c, bkcnt_vec)

        def grp(g, scnt):
            bk = g >> 3
            cntg = jnp.sum(jnp.where(lanes == bk, bkcnt_vec, 0))
            return group_step(
                tab_hbm, vlo + g * GW, GW, bk * BCAP, cntg, scnt
            )

        ngrp = jnp.where(wid == NW - 1, 62, NG)
        scnt = lax.fori_loop(0, ngrp, grp, jnp.int32(0))

        @pl.when(wid == NW - 1)
        def _():
            c7 = jnp.sum(jnp.where(lanes == NBKT - 1, bkcnt_vec, 0))
            s3 = group_step(
                tail_hbm, V - 64, 64, (NBKT - 1) * BCAP, c7, scnt, tail=True
            )
            flush(s3)

        @pl.when(wid < NW - 1)
        def _():
            flush(scnt)

    pltpu.sync_copy(cidx_hbm, idxb_v.at[pl.ds(0, B)])
    cnt = scan_chunk(B, 0, jnp.int32(0))
    run_pass(ctT_hbm, ctail_hbm, cnt)

    cnt = jnp.int32(0)
    for ch in range(B * CTX // XCHUNK):
        pltpu.sync_copy(xidx_hbm.at[pl.ds(ch * XCHUNK, XCHUNK)], idxb_v)
        cnt = scan_chunk(XCHUNK, B + ch * XCHUNK, cnt)
    run_pass(xtT_hbm, xtail_hbm, cnt)


RPW = B // NW
C2 = 64
NCH2 = RPW // C2
FPC = C2 * CTX
OPW = RPW * CTX


@functools.partial(
    pl.kernel,
    out_type=jax.ShapeDtypeStruct((B * CTX,), jnp.float32),
    mesh=_mesh,
    scratch_types=[
        pltpu.VMEM((C2, 128), jnp.float32),
        pltpu.VMEM((FPC, 128), jnp.float32),
        pltpu.VMEM((OPW,), jnp.int32),
        pltpu.VMEM((OPW,), jnp.float32),
        pltpu.SemaphoreType.DMA,
    ],
    compiler_params=pltpu.CompilerParams(needs_layout_passes=False),
)
def _dots(rows_hbm, mask_hbm, out_hbm, crows_v, xrows_v, mask_v, outb_v, sem):
    wid = lax.axis_index("s") * NC + lax.axis_index("c")
    obase = wid * OPW
    pltpu.sync_copy(mask_hbm.at[pl.ds(obase, OPW)], mask_v)
    lane = lax.iota(jnp.int32, L)
    GROUP = 8
    NVR = GROUP * CTX // L

    for chunk in range(NCH2):
        h1 = pltpu.async_copy(
            rows_hbm.at[pl.ds(wid * RPW + chunk * C2, C2)], crows_v, sem
        )
        h2 = pltpu.async_copy(
            rows_hbm.at[pl.ds(B + obase + chunk * FPC, FPC)], xrows_v, sem
        )
        h1.wait()
        h2.wait()
        dbase = chunk * FPC

        def dot_body(g, _):
            res = [jnp.zeros((L,), jnp.float32) for _ in range(NVR)]
            for r in range(GROUP):
                i = g * GROUP + r
                cvecs = [crows_v[i, pl.ds(k * L, L)] for k in range(D // L)]
                for j in range(CTX):
                    flat = i * CTX + j
                    pos = r * CTX + j
                    acc = cvecs[0] * xrows_v[flat, pl.ds(0, L)]
                    for k in range(1, D // L):
                        acc = acc + cvecs[k] * xrows_v[flat, pl.ds(k * L, L)]
                    s = jnp.sum(acc)
                    res[pos // L] = jnp.where(lane == (pos % L), s, res[pos // L])
            for t in range(NVR):
                m = mask_v[pl.ds(dbase + g * (GROUP * CTX) + t * L, L)]
                sig = 1.0 / (1.0 + jnp.exp(-res[t]))
                outb_v[pl.ds(dbase + g * (GROUP * CTX) + t * L, L)] = (
                    jnp.where(m == 0, 0.0, sig)
                )
            return 0

        lax.fori_loop(0, C2 // GROUP, dot_body, 0)

    pltpu.sync_copy(outb_v, out_hbm.at[pl.ds(obase, OPW)])


def kernel(center, context, mask, center_table, context_table):
    ctT = center_table.T
    xtT = context_table.T
    rows = _extract(
        center.reshape(-1),
        context.reshape(-1),
        ctT,
        xtT,
        ctT[:, V - 64:],
        xtT[:, V - 64:],
    )
    out = _dots(rows, mask.reshape(-1))
    return out.reshape(B, CTX)
